# baseline (speedup 1.0000x reference)
import functools

import jax
import jax.numpy as jnp
from jax.experimental import pallas as pl
from jax.experimental.pallas import tpu as pltpu

_M = 4096
_K = 512
_N = 65536
_BN = 512
_STEPS = _N // _BN
_SLOTS = 4


def _mm_kernel(a_ref, b_ref, o_hbm, stage_ref, abf_ref, sems):
    j = pl.program_id(0)
    s = jax.lax.rem(j, _SLOTS)

    @pl.when(j == 0)
    def _cast_a_once():
        abf_ref[...] = a_ref[...].astype(jnp.bfloat16)

    @pl.when(j >= _SLOTS)
    def _wait_prev():
        pltpu.make_async_copy(
            stage_ref.at[s],
            o_hbm.at[:, pl.ds((j - _SLOTS) * _BN, _BN)],
            sems.at[s],
        ).wait()

    b = b_ref[...].astype(jnp.bfloat16)
    stage_ref[s] = jnp.dot(abf_ref[...], b, preferred_element_type=jnp.float32)
    pltpu.make_async_copy(
        stage_ref.at[s],
        o_hbm.at[:, pl.ds(j * _BN, _BN)],
        sems.at[s],
    ).start()

    @pl.when(j == _STEPS - 1)
    def _drain():
        for k in range(_SLOTS):
            slot = jax.lax.rem(j - k, _SLOTS)
            pltpu.make_async_copy(
                stage_ref.at[slot],
                o_hbm.at[:, pl.ds((j - k) * _BN, _BN)],
                sems.at[slot],
            ).wait()


@functools.partial(jax.jit, static_argnames=())
def kernel(query, queue):
    return pl.pallas_call(
        _mm_kernel,
        grid=(_STEPS,),
        in_specs=[
            pl.BlockSpec((_M, _K), lambda j: (0, 0)),
            pl.BlockSpec((_K, _BN), lambda j: (0, j)),
        ],
        out_specs=pl.BlockSpec(memory_space=pl.ANY),
        out_shape=jax.ShapeDtypeStruct((_M, _N), jnp.float32),
        scratch_shapes=[
            pltpu.VMEM((_SLOTS, _M, _BN), jnp.float32),
            pltpu.VMEM((_M, _K), jnp.bfloat16),
            pltpu.SemaphoreType.DMA((_SLOTS,)),
        ],
        compiler_params=pltpu.CompilerParams(
            dimension_semantics=("arbitrary",),
            vmem_limit_bytes=63 * 1024 * 1024,
        ),
    )(query, queue)


# speedup vs baseline: 1.0011x; 1.0011x over previous
import functools

import jax
import jax.numpy as jnp
from jax.experimental import pallas as pl
from jax.experimental.pallas import tpu as pltpu

_M = 4096
_K = 512
_N = 65536
_BN = 1280


def _mm_kernel(a_ref, b_ref, o_ref):
    a = a_ref[...].astype(jnp.bfloat16)
    b = b_ref[...].astype(jnp.bfloat16)
    o_ref[...] = jnp.dot(a, b, preferred_element_type=jnp.float32)


@functools.partial(jax.jit, static_argnames=())
def kernel(query, queue):
    grid = (pl.cdiv(_N, _BN),)
    return pl.pallas_call(
        _mm_kernel,
        grid=grid,
        in_specs=[
            pl.BlockSpec((_M, _K), lambda j: (0, 0)),
            pl.BlockSpec((_K, _BN), lambda j: (0, j)),
        ],
        out_specs=pl.BlockSpec((_M, _BN), lambda j: (0, j)),
        out_shape=jax.ShapeDtypeStruct((_M, _N), jnp.float32),
        compiler_params=pltpu.CompilerParams(
            dimension_semantics=("arbitrary",),
            vmem_limit_bytes=63 * 1024 * 1024,
        ),
    )(query, queue)
